# Initial kernel scaffold; baseline (speedup 1.0000x reference)
import jax
import jax.numpy as jnp
from jax.experimental import pallas as pl


def kernel(input, weights):
    raise NotImplementedError("write your pallas kernel here")


# speedup vs baseline: 1.4770x; 1.4770x over previous
import functools

import jax
import jax.numpy as jnp
from jax import lax
from jax.experimental import pallas as pl
from jax.experimental.pallas import tpu as pltpu
from jax.experimental.pallas import tpu_sc as plsc

_PAD = 1


def _positions_body(inp_ref, out_ref):
    x = inp_ref[...]
    m = (x != _PAD).astype(jnp.int32)
    bsz, s = x.shape
    c = m
    k = 1
    while k < s:
        c = c + jnp.concatenate(
            [jnp.zeros((bsz, k), jnp.int32), c[:, : s - k]], axis=1)
        k *= 2
    out_ref[...] = c * m + _PAD


def _compute_positions(inp):
    return pl.pallas_call(
        _positions_body,
        out_shape=jax.ShapeDtypeStruct(inp.shape, jnp.int32),
    )(inp)


@functools.lru_cache(maxsize=None)
def _make_gather(B, D):
    info = plsc.get_sparse_core_info()
    NC, NS = info.num_cores, info.num_subcores
    NW = NC * NS
    b_per_w = B // NW
    CHUNK = 32
    n_chunks = b_per_w // CHUNK

    mesh = plsc.VectorSubcoreMesh(core_axis_name="c", subcore_axis_name="s")

    @functools.partial(
        pl.kernel,
        mesh=mesh,
        out_type=jax.ShapeDtypeStruct((B, D), jnp.float32),
        scratch_types=[
            pltpu.VMEM((b_per_w,), jnp.int32),
            pltpu.VMEM((CHUNK, D), jnp.float32),
            pltpu.SemaphoreType.DMA,
        ],
    )
    def gather_k(table_hbm, pos_hbm, out_hbm, idx_v, buf, gsem):
        wid = lax.axis_index("s") * NC + lax.axis_index("c")
        base = wid * b_per_w
        pltpu.sync_copy(pos_hbm.at[pl.ds(base, b_per_w)], idx_v)

        def body(c, carry):
            off = c * CHUNK
            pltpu.async_copy(
                table_hbm.at[idx_v.at[pl.ds(off, CHUNK)]], buf, gsem
            ).wait()
            pltpu.sync_copy(buf, out_hbm.at[pl.ds(base + off, CHUNK)])
            return carry

        lax.fori_loop(0, n_chunks, body, 0)

    return gather_k


def kernel(input, weights):
    inp = input.astype(jnp.int32)
    bsz, seq_len = inp.shape
    d = weights.shape[1]
    positions = _compute_positions(inp)
    flat = _make_gather(bsz * seq_len, d)(weights, positions.reshape(-1))
    return flat.reshape(bsz, seq_len, d)
